# baseline (speedup 1.0000x reference)
import functools
import math

import jax
import jax.numpy as jnp
import numpy as np
from jax import lax
from jax.experimental import pallas as pl
from jax.experimental.pallas import tpu as pltpu
from jax.experimental.pallas import tpu_sc as plsc

VOCAB = 1000000
D = 64
MAXTOK = 20
B = 1024
S = 50
EPS = 1e-5

NUM_CORES = 2
NUM_SUBCORES = 16
NUM_WORKERS = NUM_CORES * NUM_SUBCORES
N_EVENTS = B * S
EV_PER_WORKER = N_EVENTS // NUM_WORKERS
EV_PER_CHUNK = 32
CHUNKS = EV_PER_WORKER // EV_PER_CHUNK
ROWS_PER_CHUNK = EV_PER_CHUNK * MAXTOK
GATHER_ROWS = 128
N_GATHERS = ROWS_PER_CHUNK // GATHER_ROWS
NVREG = D // 16


def _mean_pe():
    position = np.arange(MAXTOK, dtype=np.float64)[:, None]
    div_term = np.exp(
        np.arange(0, D, 2, dtype=np.float64) * (-math.log(10000.0) / D))
    pe = np.zeros((MAXTOK, D), dtype=np.float64)
    pe[:, 0::2] = np.sin(position * div_term)
    pe[:, 1::2] = np.cos(position * div_term)
    return pe.mean(axis=0).astype(np.float32)


_MPE = _mean_pe()


def _lane_sum(v):
    for sh in (8, 4, 2, 1):
        perm = lax.iota(jnp.int32, 16) ^ sh
        shuf = lax.gather(
            v, perm[:, None],
            dimension_numbers=lax.GatherDimensionNumbers(
                offset_dims=(), collapsed_slice_dims=(0,),
                start_index_map=(0,)),
            slice_sizes=(1,),
            mode=lax.GatherScatterMode.PROMISE_IN_BOUNDS)
        v = v + shuf
    return v


def _rsqrt(x):
    i = lax.bitcast_convert_type(x, jnp.int32)
    y = lax.bitcast_convert_type(
        jnp.int32(0x5F3759DF) - (i >> 1), jnp.float32)
    for _ in range(3):
        y = y * (1.5 - 0.5 * x * y * y)
    return y


CONV_UNITS = 7813
VOCAB_PAD = CONV_UNITS * 128


def _conv_body(tT_h, out_h, bufs, obufs, sems, osems):
    wid = lax.axis_index("s") * NUM_CORES + lax.axis_index("c")
    base = 244 * wid + jnp.minimum(wid, 5)
    cnt = 244 + (wid < 5).astype(jnp.int32)

    def unit_off(i):
        return pl.multiple_of((base + i) * 128, 128)

    def in_copy(i, b):
        return pltpu.make_async_copy(
            tT_h.at[pl.ds(0, D), pl.ds(unit_off(i), 128)], bufs[b], sems[b])

    def out_copy(i, b):
        return pltpu.make_async_copy(
            obufs[b], out_h.at[pl.ds(unit_off(i) * D, 128 * D)], osems[b])

    iota0 = lax.iota(jnp.int32, 16)
    iotas = [iota0 + 16 * k for k in range(NVREG)]

    nbuf = len(bufs)
    for b in range(nbuf):
        in_copy(b, b).start()

    def phase(i, b):
        in_copy(i, b).wait()

        @pl.when(i >= nbuf)
        def _():
            out_copy(i, b).wait()

        for v in range(128):
            col = (iota0 + v) & 127
            c64 = col << 6
            for k in range(NVREG):
                val = plsc.load_gather(bufs[b], [iotas[k], col])
                plsc.store_scatter(obufs[b], [c64 + iotas[k]], val)

        @pl.when(i + nbuf < cnt)
        def _():
            in_copy(i + nbuf, b).start()

        out_copy(i, b).start()

    def j_body(j, carry):
        for b in range(nbuf):
            i = nbuf * j + b

            @pl.when(i < cnt)
            def _():
                phase(i, b)
        return carry

    lax.fori_loop(0, 62, j_body, 0, unroll=False)
    for b in range(nbuf):
        out_copy(0, b).wait()


def _conv_entry(tT_h, out_h, b0, b1, b2, b3, o0, o1, o2, o3, s0, s1, s2, s3,
                t0, t1, t2, t3):
    _conv_body(tT_h, out_h, (b0, b1, b2, b3), (o0, o1, o2, o3),
               (s0, s1, s2, s3), (t0, t1, t2, t3))


_conv_kernel = functools.partial(
    pl.kernel,
    out_type=jax.ShapeDtypeStruct((VOCAB_PAD * D,), jnp.float32),
    mesh=plsc.VectorSubcoreMesh(core_axis_name="c", subcore_axis_name="s"),
    compiler_params=pltpu.CompilerParams(
        use_tc_tiling_on_sc=True, needs_layout_passes=False),
    scratch_types=(
        [pltpu.VMEM((D, 128), jnp.float32)] * 4
        + [pltpu.VMEM((128 * D,), jnp.float32)] * 4
        + [pltpu.SemaphoreType.DMA] * 8
    ),
)(_conv_entry)


_TCV = 1024


def _tc_conv_body(tT_ref, out_ref):
    x = tT_ref[...]
    xt = jnp.transpose(x).reshape(_TCV // 2, 2, D)
    out_ref[:, 0:D] = xt[:, 0, :]
    out_ref[:, D:2 * D] = xt[:, 1, :]


_tc_conv = pl.pallas_call(
    _tc_conv_body,
    grid=(pl.cdiv(VOCAB, _TCV),),
    in_specs=[pl.BlockSpec((D, _TCV), lambda i: (0, i))],
    out_specs=pl.BlockSpec((_TCV // 2, 2 * D), lambda i: (i, 0)),
    out_shape=jax.ShapeDtypeStruct((VOCAB // 2, 2 * D), jnp.float32),
)


def _sc_body(table_h, ids_h, aux_h, out_h, idx_v, rows_v, out_v, aux_v, sem):
    wid = lax.axis_index("s") * NUM_CORES + lax.axis_index("c")

    pltpu.sync_copy(aux_h, aux_v)
    mpe = [aux_v[pl.ds(16 * k, 16)] for k in range(NVREG)]
    gam = [aux_v[pl.ds(D + 16 * k, 16)] for k in range(NVREG)]
    bet = [aux_v[pl.ds(2 * D + 16 * k, 16)] for k in range(NVREG)]

    inv_tok = jnp.float32(1.0 / MAXTOK)
    inv_d = jnp.float32(1.0 / D)

    def chunk_body(c, carry):
        g = wid * CHUNKS + c
        pltpu.sync_copy(ids_h.at[pl.ds(g * ROWS_PER_CHUNK, ROWS_PER_CHUNK)],
                        idx_v.at[pl.ds(0, ROWS_PER_CHUNK)])
        copies = []
        for j in range(N_GATHERS):
            copies.append(pltpu.async_copy(
                table_h.at[idx_v.at[pl.ds(j * GATHER_ROWS, GATHER_ROWS)]],
                rows_v.at[pl.ds(j * GATHER_ROWS, GATHER_ROWS)],
                sem))
        for cp in copies:
            cp.wait()

        ones = jnp.ones((16,), jnp.float32)
        zeros = jnp.zeros((16,), jnp.float32)

        def ev_body(e, carry2):
            r0 = e * MAXTOK
            v0 = idx_v[pl.ds(r0, 16)]
            v1 = idx_v[pl.ds(r0 + 16, 16)]
            m0 = jnp.where(v0 != 0, ones, zeros)
            m1 = jnp.where(v1 != 0, ones, zeros)
            acc = [jnp.zeros((16,), jnp.float32) for _ in range(NVREG)]
            for t in range(MAXTOK):
                f = m0[t] if t < 16 else m1[t - 16]
                for k in range(NVREG):
                    acc[k] = acc[k] + rows_v[r0 + t, pl.ds(16 * k, 16)] * f
            p = [acc[k] * inv_tok + mpe[k] for k in range(NVREG)]
            tot = p[0] + p[1] + p[2] + p[3]
            sq = p[0] * p[0] + p[1] * p[1] + p[2] * p[2] + p[3] * p[3]
            mu = _lane_sum(tot) * inv_d
            var = _lane_sum(sq) * inv_d - mu * mu
            rs = _rsqrt(var + EPS)
            for k in range(NVREG):
                out_v[e, pl.ds(16 * k, 16)] = (p[k] - mu) * rs * gam[k] + bet[k]
            return carry2

        lax.fori_loop(0, EV_PER_CHUNK, ev_body, 0, unroll=False)
        pltpu.sync_copy(out_v,
                        out_h.at[pl.ds(g * EV_PER_CHUNK, EV_PER_CHUNK)])
        return carry

    lax.fori_loop(0, CHUNKS, chunk_body, 0, unroll=False)


_sc_kernel = functools.partial(
    pl.kernel,
    out_type=jax.ShapeDtypeStruct((N_EVENTS, D), jnp.float32),
    mesh=plsc.VectorSubcoreMesh(core_axis_name="c", subcore_axis_name="s"),
    compiler_params=pltpu.CompilerParams(use_tc_tiling_on_sc=False),
    scratch_types=[
        pltpu.VMEM((ROWS_PER_CHUNK + 16,), jnp.int32),
        pltpu.VMEM((ROWS_PER_CHUNK, D), jnp.float32),
        pltpu.VMEM((EV_PER_CHUNK, D), jnp.float32),
        pltpu.VMEM((3 * D,), jnp.float32),
        pltpu.SemaphoreType.DMA,
    ],
)(_sc_body)


@jax.jit
def kernel(input_ids, token_table, ln_gamma, ln_beta):
    ids_flat = input_ids.reshape(-1)
    aux = jnp.concatenate([_MPE, ln_gamma, ln_beta])
    tbl_lin = _conv_kernel(token_table.T)
    out = _sc_kernel(tbl_lin.reshape(VOCAB_PAD, D), ids_flat, aux)
    return out.reshape(B, S, D)


# speedup vs baseline: 1.3445x; 1.3445x over previous
import functools
import math

import jax
import jax.numpy as jnp
import numpy as np
from jax import lax
from jax.experimental import pallas as pl
from jax.experimental.pallas import tpu as pltpu
from jax.experimental.pallas import tpu_sc as plsc

VOCAB = 1000000
D = 64
MAXTOK = 20
B = 1024
S = 50
EPS = 1e-5

NUM_CORES = 2
NUM_SUBCORES = 16
NUM_WORKERS = NUM_CORES * NUM_SUBCORES
N_EVENTS = B * S
EV_PER_WORKER = N_EVENTS // NUM_WORKERS
EV_PER_CHUNK = 32
CHUNKS = EV_PER_WORKER // EV_PER_CHUNK
ROWS_PER_CHUNK = EV_PER_CHUNK * MAXTOK
GATHER_ROWS = 128
N_GATHERS = ROWS_PER_CHUNK // GATHER_ROWS
NVREG = D // 16


def _mean_pe():
    position = np.arange(MAXTOK, dtype=np.float64)[:, None]
    div_term = np.exp(
        np.arange(0, D, 2, dtype=np.float64) * (-math.log(10000.0) / D))
    pe = np.zeros((MAXTOK, D), dtype=np.float64)
    pe[:, 0::2] = np.sin(position * div_term)
    pe[:, 1::2] = np.cos(position * div_term)
    return pe.mean(axis=0).astype(np.float32)


_MPE = _mean_pe()


def _lane_sum(v):
    for sh in (8, 4, 2, 1):
        perm = lax.iota(jnp.int32, 16) ^ sh
        shuf = lax.gather(
            v, perm[:, None],
            dimension_numbers=lax.GatherDimensionNumbers(
                offset_dims=(), collapsed_slice_dims=(0,),
                start_index_map=(0,)),
            slice_sizes=(1,),
            mode=lax.GatherScatterMode.PROMISE_IN_BOUNDS)
        v = v + shuf
    return v


def _rsqrt(x):
    i = lax.bitcast_convert_type(x, jnp.int32)
    y = lax.bitcast_convert_type(
        jnp.int32(0x5F3759DF) - (i >> 1), jnp.float32)
    for _ in range(3):
        y = y * (1.5 - 0.5 * x * y * y)
    return y


CONV_UNITS = 7813
VOCAB_PAD = CONV_UNITS * 128


def _conv_body(tT_h, out_h, bufs, obufs, sems, osems):
    wid = lax.axis_index("s") * NUM_CORES + lax.axis_index("c")
    base = 244 * wid + jnp.minimum(wid, 5)
    cnt = 244 + (wid < 5).astype(jnp.int32)

    def unit_off(i):
        return pl.multiple_of((base + i) * 128, 128)

    def in_copy(i, b):
        return pltpu.make_async_copy(
            tT_h.at[pl.ds(0, D), pl.ds(unit_off(i), 128)], bufs[b], sems[b])

    def out_copy(i, b):
        return pltpu.make_async_copy(
            obufs[b], out_h.at[pl.ds(unit_off(i) * D, 128 * D)], osems[b])

    iota0 = lax.iota(jnp.int32, 16)
    iotas = [iota0 + 16 * k for k in range(NVREG)]

    nbuf = len(bufs)
    for b in range(nbuf):
        in_copy(b, b).start()

    def phase(i, b):
        in_copy(i, b).wait()

        @pl.when(i >= nbuf)
        def _():
            out_copy(i, b).wait()

        for v in range(128):
            col = (iota0 + v) & 127
            c64 = col << 6
            for k in range(NVREG):
                val = plsc.load_gather(bufs[b], [iotas[k], col])
                plsc.store_scatter(obufs[b], [c64 + iotas[k]], val)

        @pl.when(i + nbuf < cnt)
        def _():
            in_copy(i + nbuf, b).start()

        out_copy(i, b).start()

    def j_body(j, carry):
        for b in range(nbuf):
            i = nbuf * j + b

            @pl.when(i < cnt)
            def _():
                phase(i, b)
        return carry

    lax.fori_loop(0, 62, j_body, 0, unroll=False)
    for b in range(nbuf):
        out_copy(0, b).wait()


def _conv_entry(tT_h, out_h, b0, b1, b2, b3, o0, o1, o2, o3, s0, s1, s2, s3,
                t0, t1, t2, t3):
    _conv_body(tT_h, out_h, (b0, b1, b2, b3), (o0, o1, o2, o3),
               (s0, s1, s2, s3), (t0, t1, t2, t3))


_conv_kernel = functools.partial(
    pl.kernel,
    out_type=jax.ShapeDtypeStruct((VOCAB_PAD * D,), jnp.float32),
    mesh=plsc.VectorSubcoreMesh(core_axis_name="c", subcore_axis_name="s"),
    compiler_params=pltpu.CompilerParams(
        use_tc_tiling_on_sc=True, needs_layout_passes=False),
    scratch_types=(
        [pltpu.VMEM((D, 128), jnp.float32)] * 4
        + [pltpu.VMEM((128 * D,), jnp.float32)] * 4
        + [pltpu.SemaphoreType.DMA] * 8
    ),
)(_conv_entry)


_TCV = 1024


def _tc_conv_body(tT_ref, out_ref):
    x = tT_ref[...]
    xt = jnp.transpose(x).reshape(_TCV // 2, 2, D)
    out_ref[:, 0:D] = xt[:, 0, :]
    out_ref[:, D:2 * D] = xt[:, 1, :]


_tc_conv = pl.pallas_call(
    _tc_conv_body,
    grid=(pl.cdiv(VOCAB, _TCV),),
    in_specs=[pl.BlockSpec((D, _TCV), lambda i: (0, i))],
    out_specs=pl.BlockSpec((_TCV // 2, 2 * D), lambda i: (i, 0)),
    out_shape=jax.ShapeDtypeStruct((VOCAB // 2, 2 * D), jnp.float32),
)


def _sc_body(table_h, ids_h, aux_h, out_h, idxs, rowss, outs, aux_v, isems,
             gsems, osems):
    wid = lax.axis_index("s") * NUM_CORES + lax.axis_index("c")

    pltpu.sync_copy(aux_h, aux_v)
    mpe = [aux_v[pl.ds(16 * k, 16)] for k in range(NVREG)]
    gam = [aux_v[pl.ds(D + 16 * k, 16)] for k in range(NVREG)]
    bet = [aux_v[pl.ds(2 * D + 16 * k, 16)] for k in range(NVREG)]

    inv_tok = jnp.float32(1.0 / MAXTOK)
    inv_d = jnp.float32(1.0 / D)
    ones = jnp.ones((16,), jnp.float32)
    zeros = jnp.zeros((16,), jnp.float32)

    def ids_copy(c, b):
        g = wid * CHUNKS + c
        return pltpu.make_async_copy(
            ids_h.at[pl.ds(g * ROWS_PER_CHUNK, ROWS_PER_CHUNK)],
            idxs[b].at[pl.ds(0, ROWS_PER_CHUNK)], isems[b])

    def gather(j, b):
        return pltpu.make_async_copy(
            table_h.at[idxs[b].at[pl.ds(j * GATHER_ROWS, GATHER_ROWS)]],
            rowss[b].at[pl.ds(j * GATHER_ROWS, GATHER_ROWS)], gsems[b])

    def out_copy(c, b):
        g = wid * CHUNKS + c
        return pltpu.make_async_copy(
            outs[b], out_h.at[pl.ds(g * EV_PER_CHUNK, EV_PER_CHUNK)],
            osems[b])

    ids_copy(0, 0).start()
    ids_copy(1, 1).start()
    ids_copy(0, 0).wait()
    for j in range(N_GATHERS):
        gather(j, 0).start()

    def compute(c, b):
        idx_v, rows_v, out_v = idxs[b], rowss[b], outs[b]

        def ev_body(e, carry2):
            r0 = e * MAXTOK
            v0 = idx_v[pl.ds(r0, 16)]
            v1 = idx_v[pl.ds(r0 + 16, 16)]
            m0 = jnp.where(v0 != 0, ones, zeros)
            m1 = jnp.where(v1 != 0, ones, zeros)
            acc = [jnp.zeros((16,), jnp.float32) for _ in range(NVREG)]
            for t in range(MAXTOK):
                f = m0[t] if t < 16 else m1[t - 16]
                for k in range(NVREG):
                    acc[k] = acc[k] + rows_v[r0 + t, pl.ds(16 * k, 16)] * f
            p = [acc[k] * inv_tok + mpe[k] for k in range(NVREG)]
            tot = p[0] + p[1] + p[2] + p[3]
            sq = p[0] * p[0] + p[1] * p[1] + p[2] * p[2] + p[3] * p[3]
            mu = _lane_sum(tot) * inv_d
            var = _lane_sum(sq) * inv_d - mu * mu
            rs = _rsqrt(var + EPS)
            for k in range(NVREG):
                out_v[e, pl.ds(16 * k, 16)] = (p[k] - mu) * rs * gam[k] + bet[k]
            return carry2

        lax.fori_loop(0, EV_PER_CHUNK, ev_body, 0, unroll=False)

    def phase(c, b):
        @pl.when(c + 1 < CHUNKS)
        def _():
            ids_copy(c + 1, 1 - b).wait()
            for j in range(N_GATHERS):
                gather(j, 1 - b).start()

        for j in range(N_GATHERS):
            gather(j, b).wait()

        @pl.when(c >= 2)
        def _():
            out_copy(c, b).wait()

        compute(c, b)

        @pl.when(c + 2 < CHUNKS)
        def _():
            ids_copy(c + 2, b).start()

        out_copy(c, b).start()

    def j_body(j, carry):
        for b in range(2):
            phase(2 * j + b, b)
        return carry

    lax.fori_loop(0, CHUNKS // 2, j_body, 0, unroll=False)
    for b in range(2):
        out_copy(0, b).wait()


def _sc_entry(table_h, ids_h, aux_h, out_h, i0, i1, r0, r1, o0, o1, aux_v,
              is0, is1, gs0, gs1, os0, os1):
    _sc_body(table_h, ids_h, aux_h, out_h, (i0, i1), (r0, r1), (o0, o1),
             aux_v, (is0, is1), (gs0, gs1), (os0, os1))


_sc_kernel = functools.partial(
    pl.kernel,
    out_type=jax.ShapeDtypeStruct((N_EVENTS, D), jnp.float32),
    mesh=plsc.VectorSubcoreMesh(core_axis_name="c", subcore_axis_name="s"),
    compiler_params=pltpu.CompilerParams(use_tc_tiling_on_sc=False),
    scratch_types=(
        [pltpu.VMEM((ROWS_PER_CHUNK + 16,), jnp.int32)] * 2
        + [pltpu.VMEM((ROWS_PER_CHUNK, D), jnp.float32)] * 2
        + [pltpu.VMEM((EV_PER_CHUNK, D), jnp.float32)] * 2
        + [pltpu.VMEM((3 * D,), jnp.float32)]
        + [pltpu.SemaphoreType.DMA] * 6
    ),
)(_sc_entry)


@jax.jit
def kernel(input_ids, token_table, ln_gamma, ln_beta):
    ids_flat = input_ids.reshape(-1)
    aux = jnp.concatenate([_MPE, ln_gamma, ln_beta])
    out = _sc_kernel(token_table, ids_flat, aux)
    return out.reshape(B, S, D)
